# baseline (speedup 1.0000x reference)
import functools

import jax
import jax.numpy as jnp
from jax import lax
from jax.experimental import pallas as pl
from jax.experimental.pallas import tpu as pltpu
from jax.experimental.pallas import tpu_sc as plsc

B = 16384
D = 128
LANES = 16
NUM_CORES = 2
NUM_SUBCORES = 16
NW = NUM_CORES * NUM_SUBCORES
BPW = B // NW
CHUNK = 128
NCHUNK = BPW // CHUNK
NSUPER = NCHUNK // 2
NGROUP = CHUNK // LANES


def _row_acc(hb, rb, tb, i):
    acc = (hb[i, pl.ds(0, LANES)]
           * rb[i, pl.ds(0, LANES)]
           * tb[i, pl.ds(0, LANES)])
    for k in range(1, D // LANES):
        o = k * LANES
        acc = acc + (hb[i, pl.ds(o, LANES)]
                     * rb[i, pl.ds(o, LANES)]
                     * tb[i, pl.ds(o, LANES)])
    return acc


def _perm(x, idx):
    return jnp.take_along_axis(x, idx, axis=0, mode="promise_in_bounds")


def _compute_chunk(hb, rb, tb, partials, scores, cb, lane):
    def row_body(i, carry):
        partials[i, pl.ds(0, LANES)] = _row_acc(hb, rb, tb, i)
        return carry

    lax.fori_loop(0, CHUNK, row_body, 0, unroll=8)

    def merge(a, b, s):
        mask = (lane & s) == 0
        return jnp.where(mask, a, b) + _perm(jnp.where(mask, b, a),
                                             lane ^ s)

    def group_body(g, carry):
        gb = g * LANES
        stack = []
        for j in range(LANES):
            v = partials[gb + j, pl.ds(0, LANES)]
            lvl = 0
            while stack and stack[-1][0] == lvl:
                _, pv = stack.pop()
                v = merge(pv, v, 1 << lvl)
                lvl += 1
            stack.append((lvl, v))
        scores[pl.ds(cb + gb, LANES)] = stack[0][1]
        return carry

    lax.fori_loop(0, NGROUP, group_body, 0)


def _sc_kernel(head_hbm, rel_hbm, tail_hbm, ent_hbm, relemb_hbm, out_hbm,
               hidx, ridx, tidx, scores, partials, bufs, sems):
    wid = lax.axis_index("s") * NUM_CORES + lax.axis_index("c")
    base = wid * BPW

    idx_cps = (
        pltpu.make_async_copy(head_hbm.at[pl.ds(base, BPW)], hidx, sems[0][0]),
        pltpu.make_async_copy(rel_hbm.at[pl.ds(base, BPW)], ridx, sems[0][1]),
        pltpu.make_async_copy(tail_hbm.at[pl.ds(base, BPW)], tidx, sems[0][2]),
    )
    for c in idx_cps:
        c.start()
    for c in idx_cps:
        c.wait()

    lane = lax.iota(jnp.int32, LANES)

    def copies(ci, bset, sset):
        cb = ci * CHUNK
        hb, rb, tb = bset
        sh, sr, st = sset
        return (
            pltpu.make_async_copy(ent_hbm.at[hidx.at[pl.ds(cb, CHUNK)]], hb, sh),
            pltpu.make_async_copy(relemb_hbm.at[ridx.at[pl.ds(cb, CHUNK)]], rb, sr),
            pltpu.make_async_copy(ent_hbm.at[tidx.at[pl.ds(cb, CHUNK)]], tb, st),
        )

    def start(ci, bset, sset):
        for c in copies(ci, bset, sset):
            c.start()

    def wait(ci, bset, sset):
        for c in copies(ci, bset, sset):
            c.wait()

    start(0, bufs[0], sems[0])

    def superstep(t, carry):
        c0 = 2 * t
        start(c0 + 1, bufs[1], sems[1])
        wait(c0, bufs[0], sems[0])
        _compute_chunk(bufs[0][0], bufs[0][1], bufs[0][2],
                       partials, scores, c0 * CHUNK, lane)

        @pl.when(t + 1 < NSUPER)
        def _():
            start(c0 + 2, bufs[0], sems[0])

        wait(c0 + 1, bufs[1], sems[1])
        _compute_chunk(bufs[1][0], bufs[1][1], bufs[1][2],
                       partials, scores, (c0 + 1) * CHUNK, lane)
        return carry

    lax.fori_loop(0, NSUPER, superstep, 0)
    pltpu.sync_copy(scores, out_hbm.at[pl.ds(base, BPW)])


@functools.partial(
    pl.kernel,
    mesh=plsc.VectorSubcoreMesh(core_axis_name="c", subcore_axis_name="s"),
    out_type=jax.ShapeDtypeStruct((B,), jnp.float32),
    scratch_types=[
        pltpu.VMEM((BPW,), jnp.int32),
        pltpu.VMEM((BPW,), jnp.int32),
        pltpu.VMEM((BPW,), jnp.int32),
        pltpu.VMEM((BPW,), jnp.float32),
        pltpu.VMEM((CHUNK, LANES), jnp.float32),
    ] + [pltpu.VMEM((CHUNK, D), jnp.float32) for _ in range(6)]
      + [pltpu.SemaphoreType.DMA for _ in range(6)],
)
def _distmult_sc(head_hbm, rel_hbm, tail_hbm, ent_hbm, relemb_hbm, out_hbm,
                 hidx, ridx, tidx, scores, partials,
                 hb0, rb0, tb0, hb1, rb1, tb1,
                 sh0, sr0, st0, sh1, sr1, st1):
    _sc_kernel(head_hbm, rel_hbm, tail_hbm, ent_hbm, relemb_hbm, out_hbm,
               hidx, ridx, tidx, scores, partials,
               ((hb0, rb0, tb0), (hb1, rb1, tb1)),
               ((sh0, sr0, st0), (sh1, sr1, st1)))


def kernel(head, relation, tail, entity_emb, relation_emb):
    head = head.astype(jnp.int32)
    relation = relation.astype(jnp.int32)
    tail = tail.astype(jnp.int32)
    return _distmult_sc(head, relation, tail, entity_emb, relation_emb)


# speedup vs baseline: 1.0403x; 1.0403x over previous
import functools

import jax
import jax.numpy as jnp
from jax import lax
from jax.experimental import pallas as pl
from jax.experimental.pallas import tpu as pltpu
from jax.experimental.pallas import tpu_sc as plsc

B = 16384
D = 128
LANES = 16
NUM_CORES = 2
NUM_SUBCORES = 16
NW = NUM_CORES * NUM_SUBCORES
BPW = B // NW
CHUNK = 64
NCHUNK = BPW // CHUNK
NSUPER = NCHUNK // 2
NGROUP = CHUNK // LANES


def _row_acc(hb, rb, tb, i):
    acc = (hb[i, pl.ds(0, LANES)]
           * rb[i, pl.ds(0, LANES)]
           * tb[i, pl.ds(0, LANES)])
    for k in range(1, D // LANES):
        o = k * LANES
        acc = acc + (hb[i, pl.ds(o, LANES)]
                     * rb[i, pl.ds(o, LANES)]
                     * tb[i, pl.ds(o, LANES)])
    return acc


def _perm(x, idx):
    return jnp.take_along_axis(x, idx, axis=0, mode="promise_in_bounds")


def _compute_chunk(hb, rb, tb, partials, scores, cb, lane):
    def row_body(i, carry):
        partials[i, pl.ds(0, LANES)] = _row_acc(hb, rb, tb, i)
        return carry

    lax.fori_loop(0, CHUNK, row_body, 0, unroll=8)

    def merge(a, b, s):
        mask = (lane & s) == 0
        return jnp.where(mask, a, b) + _perm(jnp.where(mask, b, a),
                                             lane ^ s)

    def group_body(g, carry):
        gb = g * LANES
        stack = []
        for j in range(LANES):
            v = partials[gb + j, pl.ds(0, LANES)]
            lvl = 0
            while stack and stack[-1][0] == lvl:
                _, pv = stack.pop()
                v = merge(pv, v, 1 << lvl)
                lvl += 1
            stack.append((lvl, v))
        scores[pl.ds(cb + gb, LANES)] = stack[0][1]
        return carry

    lax.fori_loop(0, NGROUP, group_body, 0)


def _sc_kernel(head_hbm, rel_hbm, tail_hbm, ent_hbm, relemb_hbm, out_hbm,
               hidx, ridx, tidx, scores, partials, bufs, sems):
    wid = lax.axis_index("s") * NUM_CORES + lax.axis_index("c")
    base = wid * BPW

    idx_cps = (
        pltpu.make_async_copy(head_hbm.at[pl.ds(base, BPW)], hidx, sems[0][0]),
        pltpu.make_async_copy(rel_hbm.at[pl.ds(base, BPW)], ridx, sems[0][1]),
        pltpu.make_async_copy(tail_hbm.at[pl.ds(base, BPW)], tidx, sems[0][2]),
    )
    for c in idx_cps:
        c.start()
    for c in idx_cps:
        c.wait()

    lane = lax.iota(jnp.int32, LANES)

    def copies(ci, bset, sset):
        cb = ci * CHUNK
        hb, rb, tb = bset
        sh, sr, st = sset
        return (
            pltpu.make_async_copy(ent_hbm.at[hidx.at[pl.ds(cb, CHUNK)]], hb, sh),
            pltpu.make_async_copy(relemb_hbm.at[ridx.at[pl.ds(cb, CHUNK)]], rb, sr),
            pltpu.make_async_copy(ent_hbm.at[tidx.at[pl.ds(cb, CHUNK)]], tb, st),
        )

    def start(ci, bset, sset):
        for c in copies(ci, bset, sset):
            c.start()

    def wait(ci, bset, sset):
        for c in copies(ci, bset, sset):
            c.wait()

    start(0, bufs[0], sems[0])

    def superstep(t, carry):
        c0 = 2 * t
        start(c0 + 1, bufs[1], sems[1])
        wait(c0, bufs[0], sems[0])
        _compute_chunk(bufs[0][0], bufs[0][1], bufs[0][2],
                       partials, scores, c0 * CHUNK, lane)

        @pl.when(t + 1 < NSUPER)
        def _():
            start(c0 + 2, bufs[0], sems[0])

        wait(c0 + 1, bufs[1], sems[1])
        _compute_chunk(bufs[1][0], bufs[1][1], bufs[1][2],
                       partials, scores, (c0 + 1) * CHUNK, lane)
        return carry

    lax.fori_loop(0, NSUPER, superstep, 0)
    pltpu.sync_copy(scores, out_hbm.at[pl.ds(base, BPW)])


@functools.partial(
    pl.kernel,
    mesh=plsc.VectorSubcoreMesh(core_axis_name="c", subcore_axis_name="s"),
    out_type=jax.ShapeDtypeStruct((B,), jnp.float32),
    scratch_types=[
        pltpu.VMEM((BPW,), jnp.int32),
        pltpu.VMEM((BPW,), jnp.int32),
        pltpu.VMEM((BPW,), jnp.int32),
        pltpu.VMEM((BPW,), jnp.float32),
        pltpu.VMEM((CHUNK, LANES), jnp.float32),
    ] + [pltpu.VMEM((CHUNK, D), jnp.float32) for _ in range(6)]
      + [pltpu.SemaphoreType.DMA for _ in range(6)],
)
def _distmult_sc(head_hbm, rel_hbm, tail_hbm, ent_hbm, relemb_hbm, out_hbm,
                 hidx, ridx, tidx, scores, partials,
                 hb0, rb0, tb0, hb1, rb1, tb1,
                 sh0, sr0, st0, sh1, sr1, st1):
    _sc_kernel(head_hbm, rel_hbm, tail_hbm, ent_hbm, relemb_hbm, out_hbm,
               hidx, ridx, tidx, scores, partials,
               ((hb0, rb0, tb0), (hb1, rb1, tb1)),
               ((sh0, sr0, st0), (sh1, sr1, st1)))


def kernel(head, relation, tail, entity_emb, relation_emb):
    head = head.astype(jnp.int32)
    relation = relation.astype(jnp.int32)
    tail = tail.astype(jnp.int32)
    return _distmult_sc(head, relation, tail, entity_emb, relation_emb)


# speedup vs baseline: 1.1447x; 1.1003x over previous
import functools

import jax
import jax.numpy as jnp
from jax import lax
from jax.experimental import pallas as pl
from jax.experimental.pallas import tpu as pltpu
from jax.experimental.pallas import tpu_sc as plsc

B = 16384
D = 128
LANES = 16
NUM_CORES = 2
NUM_SUBCORES = 16
NW = NUM_CORES * NUM_SUBCORES
BPW = B // NW
CHUNK = 64
NCHUNK = BPW // CHUNK
NSUPER = NCHUNK // 2
NGROUP = CHUNK // LANES


def _row_acc(hb, rb, tb, i):
    acc = (hb[i, pl.ds(0, LANES)]
           * rb[i, pl.ds(0, LANES)]
           * tb[i, pl.ds(0, LANES)])
    for k in range(1, D // LANES):
        o = k * LANES
        acc = acc + (hb[i, pl.ds(o, LANES)]
                     * rb[i, pl.ds(o, LANES)]
                     * tb[i, pl.ds(o, LANES)])
    return acc


def _perm(x, idx):
    return jnp.take_along_axis(x, idx, axis=0, mode="promise_in_bounds")


def _compute_chunk(hb, rb, tb, partials, scores, cb, lane):
    def merge(a, b, s):
        mask = (lane & s) == 0
        return jnp.where(mask, a, b) + _perm(jnp.where(mask, b, a),
                                             lane ^ s)

    def packet_body(p, carry):
        i = p * 4
        v0 = _row_acc(hb, rb, tb, i)
        v1 = _row_acc(hb, rb, tb, i + 1)
        v2 = _row_acc(hb, rb, tb, i + 2)
        v3 = _row_acc(hb, rb, tb, i + 3)
        m = merge(merge(v0, v1, 1), merge(v2, v3, 1), 2)
        partials[p, pl.ds(0, LANES)] = m
        return carry

    lax.fori_loop(0, CHUNK // 4, packet_body, 0, unroll=2)

    def group_body(g, carry):
        pb = g * 4
        p0 = partials[pb, pl.ds(0, LANES)]
        p1 = partials[pb + 1, pl.ds(0, LANES)]
        p2 = partials[pb + 2, pl.ds(0, LANES)]
        p3 = partials[pb + 3, pl.ds(0, LANES)]
        vec = merge(merge(p0, p1, 4), merge(p2, p3, 4), 8)
        scores[pl.ds(cb + g * LANES, LANES)] = vec
        return carry

    lax.fori_loop(0, NGROUP, group_body, 0, unroll=2)


def _sc_kernel(head_hbm, rel_hbm, tail_hbm, ent_hbm, relemb_hbm, out_hbm,
               hidx, ridx, tidx, scores, partials, bufs, sems, sem_out):
    wid = lax.axis_index("s") * NUM_CORES + lax.axis_index("c")
    base = wid * BPW

    idx_a = (
        pltpu.make_async_copy(head_hbm.at[pl.ds(base, CHUNK)],
                              hidx.at[pl.ds(0, CHUNK)], sems[0][0]),
        pltpu.make_async_copy(rel_hbm.at[pl.ds(base, CHUNK)],
                              ridx.at[pl.ds(0, CHUNK)], sems[0][1]),
        pltpu.make_async_copy(tail_hbm.at[pl.ds(base, CHUNK)],
                              tidx.at[pl.ds(0, CHUNK)], sems[0][2]),
    )
    rest = BPW - CHUNK
    idx_b = (
        pltpu.make_async_copy(head_hbm.at[pl.ds(base + CHUNK, rest)],
                              hidx.at[pl.ds(CHUNK, rest)], sems[1][0]),
        pltpu.make_async_copy(rel_hbm.at[pl.ds(base + CHUNK, rest)],
                              ridx.at[pl.ds(CHUNK, rest)], sems[1][1]),
        pltpu.make_async_copy(tail_hbm.at[pl.ds(base + CHUNK, rest)],
                              tidx.at[pl.ds(CHUNK, rest)], sems[1][2]),
    )
    for c in idx_a + idx_b:
        c.start()
    for c in idx_a:
        c.wait()

    lane = lax.iota(jnp.int32, LANES)

    def copies(ci, bset, sset):
        cb = ci * CHUNK
        hb, rb, tb = bset
        sh, sr, st = sset
        return (
            pltpu.make_async_copy(ent_hbm.at[hidx.at[pl.ds(cb, CHUNK)]], hb, sh),
            pltpu.make_async_copy(relemb_hbm.at[ridx.at[pl.ds(cb, CHUNK)]], rb, sr),
            pltpu.make_async_copy(ent_hbm.at[tidx.at[pl.ds(cb, CHUNK)]], tb, st),
        )

    def start(ci, bset, sset):
        for c in copies(ci, bset, sset):
            c.start()

    def wait(ci, bset, sset):
        for c in copies(ci, bset, sset):
            c.wait()

    start(0, bufs[0], sems[0])
    for c in idx_b:
        c.wait()

    def superstep(t, carry):
        c0 = 2 * t
        start(c0 + 1, bufs[1], sems[1])
        wait(c0, bufs[0], sems[0])
        _compute_chunk(bufs[0][0], bufs[0][1], bufs[0][2],
                       partials, scores, c0 * CHUNK, lane)

        @pl.when(t + 1 < NSUPER)
        def _():
            start(c0 + 2, bufs[0], sems[0])

        wait(c0 + 1, bufs[1], sems[1])
        _compute_chunk(bufs[1][0], bufs[1][1], bufs[1][2],
                       partials, scores, (c0 + 1) * CHUNK, lane)
        pltpu.make_async_copy(
            scores.at[pl.ds(c0 * CHUNK, 2 * CHUNK)],
            out_hbm.at[pl.ds(base + c0 * CHUNK, 2 * CHUNK)],
            sem_out).start()
        return carry

    lax.fori_loop(0, NSUPER, superstep, 0)
    for _ in range(NSUPER):
        pltpu.make_async_copy(
            scores.at[pl.ds(0, 2 * CHUNK)],
            out_hbm.at[pl.ds(base, 2 * CHUNK)],
            sem_out).wait()


@functools.partial(
    pl.kernel,
    mesh=plsc.VectorSubcoreMesh(core_axis_name="c", subcore_axis_name="s"),
    out_type=jax.ShapeDtypeStruct((B,), jnp.float32),
    scratch_types=[
        pltpu.VMEM((BPW,), jnp.int32),
        pltpu.VMEM((BPW,), jnp.int32),
        pltpu.VMEM((BPW,), jnp.int32),
        pltpu.VMEM((BPW,), jnp.float32),
        pltpu.VMEM((CHUNK // 4, LANES), jnp.float32),
    ] + [pltpu.VMEM((CHUNK, D), jnp.float32) for _ in range(6)]
      + [pltpu.SemaphoreType.DMA for _ in range(7)],
)
def _distmult_sc(head_hbm, rel_hbm, tail_hbm, ent_hbm, relemb_hbm, out_hbm,
                 hidx, ridx, tidx, scores, partials,
                 hb0, rb0, tb0, hb1, rb1, tb1,
                 sh0, sr0, st0, sh1, sr1, st1, sem_out):
    _sc_kernel(head_hbm, rel_hbm, tail_hbm, ent_hbm, relemb_hbm, out_hbm,
               hidx, ridx, tidx, scores, partials,
               ((hb0, rb0, tb0), (hb1, rb1, tb1)),
               ((sh0, sr0, st0), (sh1, sr1, st1)), sem_out)


def kernel(head, relation, tail, entity_emb, relation_emb):
    head = head.astype(jnp.int32)
    relation = relation.astype(jnp.int32)
    tail = tail.astype(jnp.int32)
    return _distmult_sc(head, relation, tail, entity_emb, relation_emb)
